# Initial kernel scaffold; baseline (speedup 1.0000x reference)
import jax
import jax.numpy as jnp
from jax.experimental import pallas as pl


def kernel(x, edge_index, W, b):
    raise NotImplementedError("write your pallas kernel here")


# speedup vs baseline: 20.9387x; 20.9387x over previous
import dataclasses
import functools

import jax
import jax.numpy as jnp
from jax import lax
from jax.experimental import pallas as pl
from jax.experimental.pallas import tpu as pltpu
from jax.experimental.pallas import tpu_sc as plsc

NC = 2
NS = 16
NW = NC * NS
LANES = 16
CHUNK = 128


def _sc_mesh():
    return plsc.VectorSubcoreMesh(core_axis_name="c", subcore_axis_name="s")


def _sc_params():
    cp = pltpu.CompilerParams()
    if "needs_layout_passes" in pltpu.CompilerParams.__dataclass_fields__:
        cp = dataclasses.replace(cp, needs_layout_passes=False)
    return cp


def _pad_rows(n_nodes):
    return -(-n_nodes // (NS * 8)) * (NS * 8)


def _sc_hist(dst, n_nodes):
    n_edges = dst.shape[0]
    nchunks = n_edges // CHUNK
    nt = -(-nchunks // NW)
    n_pad = _pad_rows(n_nodes)
    hrows = n_pad // LANES

    @functools.partial(
        pl.kernel,
        out_type=jax.ShapeDtypeStruct((NW, hrows, LANES), jnp.int32),
        mesh=_sc_mesh(),
        scratch_types=[
            pltpu.VMEM((CHUNK,), jnp.int32),
            pltpu.VMEM((hrows, LANES), jnp.int32),
            pltpu.SemaphoreType.DMA,
        ],
        compiler_params=_sc_params(),
    )
    def hist(dst_hbm, out_hbm, idx_v, hist_v, sem):
        cid = lax.axis_index("c")
        sid = lax.axis_index("s")
        wid = sid * NC + cid
        zero = jnp.zeros((LANES,), jnp.int32)

        @pl.loop(0, hrows)
        def _(r):
            hist_v[r, :] = zero

        @pl.loop(0, nt)
        def _(t):
            c = wid + NW * t

            @pl.when(c < nchunks)
            def _():
                base = pl.multiple_of(c * CHUNK, CHUNK)
                pltpu.sync_copy(dst_hbm.at[pl.ds(base, CHUNK)], idx_v)
                for k in range(CHUNK // LANES):
                    v = idx_v[pl.ds(k * LANES, LANES)]
                    cnt, last = plsc.scan_count(v)
                    plsc.addupdate_scatter(
                        hist_v, [v >> 4, v & 15], cnt, mask=last)

        pltpu.sync_copy(hist_v, out_hbm.at[wid])

    return hist(dst)


def _sc_agg(hp, src, dst):
    n_nodes, d_out = hp.shape
    n_edges = src.shape[0]
    nchunks = n_edges // CHUNK
    nt = -(-nchunks // NW)
    n_pad = _pad_rows(n_nodes)
    rps = n_pad // NS

    @functools.partial(
        pl.kernel,
        out_type=jax.ShapeDtypeStruct((NC, n_pad, d_out), jnp.float32),
        mesh=_sc_mesh(),
        scratch_types=[
            pltpu.VMEM((CHUNK,), jnp.int32),
            pltpu.VMEM((CHUNK,), jnp.int32),
            pltpu.VMEM((CHUNK, d_out), jnp.float32),
            pltpu.VMEM_SHARED((n_pad, d_out), jnp.float32),
            pltpu.SemaphoreType.DMA,
        ],
    )
    def agg(hp_hbm, src_hbm, dst_hbm, out_hbm,
            sidx_v, didx_v, rows_v, agg_sh, sem):
        cid = lax.axis_index("c")
        sid = lax.axis_index("s")
        wid = sid * NC + cid
        zero = jnp.zeros((LANES,), jnp.float32)

        @pl.loop(0, CHUNK)
        def _(r):
            @pl.loop(0, d_out // LANES)
            def _(j):
                rows_v[r, pl.ds(j * LANES, LANES)] = zero

        @pl.loop(0, rps // CHUNK)
        def _(k):
            pltpu.sync_copy(rows_v,
                            agg_sh.at[pl.ds(sid * rps + k * CHUNK, CHUNK)])

        if rps % CHUNK:
            rem = rps % CHUNK
            pltpu.sync_copy(
                rows_v.at[pl.ds(0, rem)],
                agg_sh.at[pl.ds(sid * rps + (rps // CHUNK) * CHUNK, rem)])
        plsc.subcore_barrier()

        @pl.loop(0, nt)
        def _(t):
            c = wid + NW * t

            @pl.when(c < nchunks)
            def _():
                base = pl.multiple_of(c * CHUNK, CHUNK)
                pltpu.sync_copy(src_hbm.at[pl.ds(base, CHUNK)], sidx_v)
                pltpu.sync_copy(dst_hbm.at[pl.ds(base, CHUNK)], didx_v)
                pltpu.async_copy(hp_hbm.at[sidx_v], rows_v, sem).wait()
                pltpu.sync_copy(rows_v, agg_sh.at[didx_v], add=True)

        plsc.subcore_barrier()
        r0 = sid * rps
        pltpu.sync_copy(agg_sh.at[pl.ds(r0, rps)],
                        out_hbm.at[cid, pl.ds(r0, rps)])

    return agg(hp, src, dst)


def _deg_col(hist_ref, rb):
    hist = hist_ref[...].astype(jnp.float32)
    ones = jnp.ones((NW, 1), jnp.float32)
    return lax.dot_general(hist, ones, (((0,), (0,)), ((), ()))) + 1.0


def _tc_matmul_scale(x, W, histp):
    n_nodes, d_in = x.shape
    d_out = W.shape[1]
    rb = 1024

    def body(x_ref, w_ref, hist_ref, o_ref):
        deg = _deg_col(hist_ref, rb)
        h = jnp.dot(x_ref[...], w_ref[...], preferred_element_type=jnp.float32)
        o_ref[...] = h * lax.rsqrt(deg)

    return pl.pallas_call(
        body,
        grid=(pl.cdiv(n_nodes, rb),),
        in_specs=[
            pl.BlockSpec((rb, d_in), lambda i: (i, 0)),
            pl.BlockSpec((d_in, d_out), lambda i: (0, 0)),
            pl.BlockSpec((NW, rb), lambda i: (0, i)),
        ],
        out_specs=pl.BlockSpec((rb, d_out), lambda i: (i, 0)),
        out_shape=jax.ShapeDtypeStruct((n_nodes, d_out), jnp.float32),
    )(x, W, histp)


def _tc_combine(aggp, hp, histp, b):
    n_nodes, d_out = hp.shape
    rb = 1024

    def body(p0_ref, p1_ref, hp_ref, hist_ref, b_ref, o_ref):
        deg = _deg_col(hist_ref, rb)
        s = (p0_ref[...] + p1_ref[...] + hp_ref[...]) * lax.rsqrt(deg)
        o_ref[...] = jnp.maximum(s + b_ref[...], 0.0)

    return pl.pallas_call(
        body,
        grid=(pl.cdiv(n_nodes, rb),),
        in_specs=[
            pl.BlockSpec((rb, d_out), lambda i: (i, 0)),
            pl.BlockSpec((rb, d_out), lambda i: (i, 0)),
            pl.BlockSpec((rb, d_out), lambda i: (i, 0)),
            pl.BlockSpec((NW, rb), lambda i: (0, i)),
            pl.BlockSpec((1, d_out), lambda i: (0, 0)),
        ],
        out_specs=pl.BlockSpec((rb, d_out), lambda i: (i, 0)),
        out_shape=jax.ShapeDtypeStruct((n_nodes, d_out), jnp.float32),
    )(aggp[0], aggp[1], hp, histp, b.reshape(1, d_out))


def kernel(x, edge_index, W, b):
    n_nodes = x.shape[0]
    n_pad = _pad_rows(n_nodes)
    src = edge_index[0]
    dst = edge_index[1]
    histp = _sc_hist(dst, n_nodes).reshape(NW, n_pad)
    hp = _tc_matmul_scale(x, W, histp)
    aggp = _sc_agg(hp, src, dst)
    return _tc_combine(aggp, hp, histp, b)


# speedup vs baseline: 30.4444x; 1.4540x over previous
import dataclasses
import functools

import jax
import jax.numpy as jnp
from jax import lax
from jax.experimental import pallas as pl
from jax.experimental.pallas import tpu as pltpu
from jax.experimental.pallas import tpu_sc as plsc

NC = 2
NS = 16
NW = NC * NS
LANES = 16
CHUNK = 128


def _sc_mesh():
    return plsc.VectorSubcoreMesh(core_axis_name="c", subcore_axis_name="s")


def _sc_params():
    cp = pltpu.CompilerParams()
    if "needs_layout_passes" in pltpu.CompilerParams.__dataclass_fields__:
        cp = dataclasses.replace(cp, needs_layout_passes=False)
    return cp


def _pad_rows(n_nodes):
    return -(-n_nodes // (NS * 8)) * (NS * 8)


def _sc_hist(dst, n_nodes):
    n_edges = dst.shape[0]
    nchunks = n_edges // CHUNK
    nt = -(-nchunks // NW)
    n_pad = _pad_rows(n_nodes)
    hrows = n_pad // LANES

    @functools.partial(
        pl.kernel,
        out_type=jax.ShapeDtypeStruct((NW, hrows, LANES), jnp.int32),
        mesh=_sc_mesh(),
        scratch_types=[
            pltpu.VMEM((CHUNK,), jnp.int32),
            pltpu.VMEM((hrows, LANES), jnp.int32),
            pltpu.SemaphoreType.DMA,
        ],
        compiler_params=_sc_params(),
    )
    def hist(dst_hbm, out_hbm, idx_v, hist_v, sem):
        cid = lax.axis_index("c")
        sid = lax.axis_index("s")
        wid = sid * NC + cid
        zero = jnp.zeros((LANES,), jnp.int32)

        @pl.loop(0, hrows)
        def _(r):
            hist_v[r, :] = zero

        @pl.loop(0, nt)
        def _(t):
            c = wid + NW * t

            @pl.when(c < nchunks)
            def _():
                base = pl.multiple_of(c * CHUNK, CHUNK)
                pltpu.sync_copy(dst_hbm.at[pl.ds(base, CHUNK)], idx_v)
                for k in range(CHUNK // LANES):
                    v = idx_v[pl.ds(k * LANES, LANES)]
                    cnt, last = plsc.scan_count(v)
                    plsc.addupdate_scatter(
                        hist_v, [v >> 4, v & 15], cnt, mask=last)

        pltpu.sync_copy(hist_v, out_hbm.at[wid])

    return hist(dst)


def _sc_agg(hp, src, dst):
    n_nodes, d_out = hp.shape
    n_edges = src.shape[0]
    nchunks = n_edges // CHUNK
    nt = -(-nchunks // NW)
    n_pad = _pad_rows(n_nodes)
    rps = n_pad // NS

    @functools.partial(
        pl.kernel,
        out_type=jax.ShapeDtypeStruct((NC, n_pad, d_out), jnp.float32),
        mesh=_sc_mesh(),
        scratch_types=[
            pltpu.VMEM((CHUNK,), jnp.int32),
            pltpu.VMEM((CHUNK,), jnp.int32),
            pltpu.VMEM((CHUNK,), jnp.int32),
            pltpu.VMEM((CHUNK,), jnp.int32),
            pltpu.VMEM((CHUNK, d_out), jnp.float32),
            pltpu.VMEM((CHUNK, d_out), jnp.float32),
            pltpu.VMEM_SHARED((n_pad, d_out), jnp.float32),
            pltpu.SemaphoreType.DMA,
            pltpu.SemaphoreType.DMA,
            pltpu.SemaphoreType.DMA,
            pltpu.SemaphoreType.DMA,
            pltpu.SemaphoreType.DMA,
            pltpu.SemaphoreType.DMA,
        ],
        compiler_params=_sc_params(),
    )
    def agg(hp_hbm, src_hbm, dst_hbm, out_hbm,
            sidx_a, sidx_b, didx_a, didx_b, rows_a, rows_b, agg_sh,
            isem_sa, isem_sb, isem_da, isem_db, gsem_a, gsem_b):
        cid = lax.axis_index("c")
        sid = lax.axis_index("s")
        wid = sid * NC + cid
        ntw = (nchunks - wid + NW - 1) // NW
        zero = jnp.zeros((LANES,), jnp.float32)

        sidx = (sidx_a, sidx_b)
        didx = (didx_a, didx_b)
        rows = (rows_a, rows_b)
        isem_s = (isem_sa, isem_sb)
        isem_d = (isem_da, isem_db)
        gsem = (gsem_a, gsem_b)

        def chunk_base(t):
            return pl.multiple_of((wid + NW * t) * CHUNK, CHUNK)

        def issue_idx(t, b):
            base = chunk_base(t)
            pltpu.async_copy(src_hbm.at[pl.ds(base, CHUNK)], sidx[b],
                             isem_s[b])
            pltpu.async_copy(dst_hbm.at[pl.ds(base, CHUNK)], didx[b],
                             isem_d[b])

        def wait_idx(t, b):
            base = chunk_base(t)
            pltpu.make_async_copy(src_hbm.at[pl.ds(base, CHUNK)], sidx[b],
                                  isem_s[b]).wait()
            pltpu.make_async_copy(dst_hbm.at[pl.ds(base, CHUNK)], didx[b],
                                  isem_d[b]).wait()

        @pl.loop(0, CHUNK)
        def _(r):
            @pl.loop(0, d_out // LANES)
            def _(j):
                rows_a[r, pl.ds(j * LANES, LANES)] = zero

        @pl.loop(0, rps // CHUNK)
        def _(k):
            pltpu.sync_copy(rows_a,
                            agg_sh.at[pl.ds(sid * rps + k * CHUNK, CHUNK)])

        if rps % CHUNK:
            rem = rps % CHUNK
            pltpu.sync_copy(
                rows_a.at[pl.ds(0, rem)],
                agg_sh.at[pl.ds(sid * rps + (rps // CHUNK) * CHUNK, rem)])
        plsc.subcore_barrier()

        issue_idx(0, 0)
        issue_idx(1, 1)
        wait_idx(0, 0)
        pltpu.async_copy(hp_hbm.at[sidx_a], rows_a, gsem_a)

        def phase(t, b):
            @pl.when(t < ntw)
            def _():
                pltpu.make_async_copy(hp_hbm.at[sidx[b]], rows[b],
                                      gsem[b]).wait()

            @pl.when(t + 1 < ntw)
            def _():
                wait_idx(t + 1, 1 - b)
                pltpu.async_copy(hp_hbm.at[sidx[1 - b]], rows[1 - b],
                                 gsem[1 - b])

            @pl.when(t < ntw)
            def _():
                pltpu.sync_copy(rows[b], agg_sh.at[didx[b]], add=True)

            @pl.when(t + 2 < ntw)
            def _():
                issue_idx(t + 2, b)

        @pl.loop(0, (nt + 2) // 2)
        def _(u):
            phase(2 * u, 0)
            phase(2 * u + 1, 1)

        plsc.subcore_barrier()
        r0 = sid * rps
        pltpu.sync_copy(agg_sh.at[pl.ds(r0, rps)],
                        out_hbm.at[cid, pl.ds(r0, rps)])

    return agg(hp, src, dst)


def _deg_col(hist_ref, rb):
    hist = hist_ref[...].astype(jnp.float32)
    ones = jnp.ones((NW, 1), jnp.float32)
    return lax.dot_general(hist, ones, (((0,), (0,)), ((), ()))) + 1.0


def _tc_matmul_scale(x, W, histp):
    n_nodes, d_in = x.shape
    d_out = W.shape[1]
    rb = 1024

    def body(x_ref, w_ref, hist_ref, o_ref):
        deg = _deg_col(hist_ref, rb)
        h = jnp.dot(x_ref[...], w_ref[...], preferred_element_type=jnp.float32)
        o_ref[...] = h * lax.rsqrt(deg)

    return pl.pallas_call(
        body,
        grid=(pl.cdiv(n_nodes, rb),),
        in_specs=[
            pl.BlockSpec((rb, d_in), lambda i: (i, 0)),
            pl.BlockSpec((d_in, d_out), lambda i: (0, 0)),
            pl.BlockSpec((NW, rb), lambda i: (0, i)),
        ],
        out_specs=pl.BlockSpec((rb, d_out), lambda i: (i, 0)),
        out_shape=jax.ShapeDtypeStruct((n_nodes, d_out), jnp.float32),
    )(x, W, histp)


def _tc_combine(aggp, hp, histp, b):
    n_nodes, d_out = hp.shape
    rb = 1024

    def body(p0_ref, p1_ref, hp_ref, hist_ref, b_ref, o_ref):
        deg = _deg_col(hist_ref, rb)
        s = (p0_ref[...] + p1_ref[...] + hp_ref[...]) * lax.rsqrt(deg)
        o_ref[...] = jnp.maximum(s + b_ref[...], 0.0)

    return pl.pallas_call(
        body,
        grid=(pl.cdiv(n_nodes, rb),),
        in_specs=[
            pl.BlockSpec((rb, d_out), lambda i: (i, 0)),
            pl.BlockSpec((rb, d_out), lambda i: (i, 0)),
            pl.BlockSpec((rb, d_out), lambda i: (i, 0)),
            pl.BlockSpec((NW, rb), lambda i: (0, i)),
            pl.BlockSpec((1, d_out), lambda i: (0, 0)),
        ],
        out_specs=pl.BlockSpec((rb, d_out), lambda i: (i, 0)),
        out_shape=jax.ShapeDtypeStruct((n_nodes, d_out), jnp.float32),
    )(aggp[0], aggp[1], hp, histp, b.reshape(1, d_out))


def kernel(x, edge_index, W, b):
    n_nodes = x.shape[0]
    n_pad = _pad_rows(n_nodes)
    src = edge_index[0]
    dst = edge_index[1]
    histp = _sc_hist(dst, n_nodes).reshape(NW, n_pad)
    hp = _tc_matmul_scale(x, W, histp)
    aggp = _sc_agg(hp, src, dst)
    return _tc_combine(aggp, hp, histp, b)


# speedup vs baseline: 37.4294x; 1.2294x over previous
import dataclasses
import functools

import jax
import jax.numpy as jnp
from jax import lax
from jax.experimental import pallas as pl
from jax.experimental.pallas import tpu as pltpu
from jax.experimental.pallas import tpu_sc as plsc

NC = 2
NS = 16
NW = NC * NS
LANES = 16
CHUNK = 128


def _sc_mesh():
    return plsc.VectorSubcoreMesh(core_axis_name="c", subcore_axis_name="s")


def _sc_params():
    cp = pltpu.CompilerParams()
    if "needs_layout_passes" in pltpu.CompilerParams.__dataclass_fields__:
        cp = dataclasses.replace(cp, needs_layout_passes=False)
    return cp


def _pad_rows(n_nodes):
    return -(-n_nodes // (NS * 8)) * (NS * 8)


def _sc_hist(dst, n_nodes):
    n_edges = dst.shape[0]
    nchunks = n_edges // CHUNK
    nt = -(-nchunks // NW)
    n_pad = _pad_rows(n_nodes)
    hrows = n_pad // LANES

    @functools.partial(
        pl.kernel,
        out_type=jax.ShapeDtypeStruct((NW, hrows, LANES), jnp.int32),
        mesh=_sc_mesh(),
        scratch_types=[
            pltpu.VMEM((CHUNK,), jnp.int32),
            pltpu.VMEM((CHUNK,), jnp.int32),
            pltpu.VMEM((CHUNK,), jnp.int32),
            pltpu.VMEM((CHUNK,), jnp.int32),
            pltpu.VMEM((hrows, LANES), jnp.int32),
            pltpu.SemaphoreType.DMA,
            pltpu.SemaphoreType.DMA,
            pltpu.SemaphoreType.DMA,
            pltpu.SemaphoreType.DMA,
        ],
        compiler_params=_sc_params(),
    )
    def hist(dst_hbm, out_hbm, i0, i1, i2, i3, hist_v, s0, s1, s2, s3):
        cid = lax.axis_index("c")
        sid = lax.axis_index("s")
        wid = sid * NC + cid
        ntw = (nchunks - wid + NW - 1) // NW
        zero = jnp.zeros((LANES,), jnp.int32)
        idx = (i0, i1, i2, i3)
        sem = (s0, s1, s2, s3)

        @pl.loop(0, hrows)
        def _(r):
            hist_v[r, :] = zero

        def idx_src(t):
            base = pl.multiple_of((wid + NW * t) * CHUNK, CHUNK)
            return dst_hbm.at[pl.ds(base, CHUNK)]

        for b in range(4):
            pltpu.async_copy(idx_src(b), idx[b], sem[b])

        def phase(t, b):
            @pl.when(t < ntw)
            def _():
                pltpu.make_async_copy(idx_src(t), idx[b], sem[b]).wait()
                for k in range(CHUNK // LANES):
                    v = idx[b][pl.ds(k * LANES, LANES)]
                    cnt, last = plsc.scan_count(v)
                    plsc.addupdate_scatter(
                        hist_v, [v >> 4, v & 15], cnt, mask=last)

            @pl.when(t + 4 < ntw)
            def _():
                pltpu.async_copy(idx_src(t + 4), idx[b], sem[b])

        @pl.loop(0, (nt + 3) // 4)
        def _(u):
            for b in range(4):
                phase(4 * u + b, b)

        pltpu.sync_copy(hist_v, out_hbm.at[wid])

    return hist(dst)


def _sc_agg(hp, src, dst):
    n_nodes, d_out = hp.shape
    n_edges = src.shape[0]
    nchunks = n_edges // CHUNK
    nt = -(-nchunks // NW)
    n_pad = _pad_rows(n_nodes)
    rps = n_pad // NS

    @functools.partial(
        pl.kernel,
        out_type=jax.ShapeDtypeStruct((NC, n_pad, d_out), jnp.float32),
        mesh=_sc_mesh(),
        scratch_types=[
            pltpu.VMEM((CHUNK,), jnp.int32),
            pltpu.VMEM((CHUNK,), jnp.int32),
            pltpu.VMEM((CHUNK,), jnp.int32),
            pltpu.VMEM((CHUNK,), jnp.int32),
            pltpu.VMEM((CHUNK,), jnp.int32),
            pltpu.VMEM((CHUNK,), jnp.int32),
            pltpu.VMEM((CHUNK, d_out), jnp.float32),
            pltpu.VMEM((CHUNK, d_out), jnp.float32),
            pltpu.VMEM((CHUNK, d_out), jnp.float32),
            pltpu.VMEM_SHARED((n_pad, d_out), jnp.float32),
            pltpu.SemaphoreType.DMA,
            pltpu.SemaphoreType.DMA,
            pltpu.SemaphoreType.DMA,
            pltpu.SemaphoreType.DMA,
            pltpu.SemaphoreType.DMA,
            pltpu.SemaphoreType.DMA,
            pltpu.SemaphoreType.DMA,
            pltpu.SemaphoreType.DMA,
            pltpu.SemaphoreType.DMA,
        ],
        compiler_params=_sc_params(),
    )
    def agg(hp_hbm, src_hbm, dst_hbm, out_hbm,
            sx0, sx1, sx2, dx0, dx1, dx2, r0v, r1v, r2v, agg_sh,
            ss0, ss1, ss2, ds0, ds1, ds2, gs0, gs1, gs2):
        cid = lax.axis_index("c")
        sid = lax.axis_index("s")
        wid = sid * NC + cid
        ntw = (nchunks - wid + NW - 1) // NW
        zero = jnp.zeros((LANES,), jnp.float32)

        sidx = (sx0, sx1, sx2)
        didx = (dx0, dx1, dx2)
        rows = (r0v, r1v, r2v)
        isem_s = (ss0, ss1, ss2)
        isem_d = (ds0, ds1, ds2)
        gsem = (gs0, gs1, gs2)

        def chunk_base(t):
            return pl.multiple_of((wid + NW * t) * CHUNK, CHUNK)

        def issue_idx(t, b):
            base = chunk_base(t)
            pltpu.async_copy(src_hbm.at[pl.ds(base, CHUNK)], sidx[b],
                             isem_s[b])
            pltpu.async_copy(dst_hbm.at[pl.ds(base, CHUNK)], didx[b],
                             isem_d[b])

        def wait_idx(t, b):
            base = chunk_base(t)
            pltpu.make_async_copy(src_hbm.at[pl.ds(base, CHUNK)], sidx[b],
                                  isem_s[b]).wait()
            pltpu.make_async_copy(dst_hbm.at[pl.ds(base, CHUNK)], didx[b],
                                  isem_d[b]).wait()

        @pl.loop(0, CHUNK)
        def _(r):
            @pl.loop(0, d_out // LANES)
            def _(j):
                r0v[r, pl.ds(j * LANES, LANES)] = zero

        @pl.loop(0, rps // CHUNK)
        def _(k):
            pltpu.sync_copy(r0v,
                            agg_sh.at[pl.ds(sid * rps + k * CHUNK, CHUNK)])

        if rps % CHUNK:
            rem = rps % CHUNK
            pltpu.sync_copy(
                r0v.at[pl.ds(0, rem)],
                agg_sh.at[pl.ds(sid * rps + (rps // CHUNK) * CHUNK, rem)])
        plsc.subcore_barrier()

        issue_idx(0, 0)
        issue_idx(1, 1)
        wait_idx(0, 0)
        pltpu.async_copy(hp_hbm.at[sidx[0]], rows[0], gsem[0])
        wait_idx(1, 1)
        pltpu.async_copy(hp_hbm.at[sidx[1]], rows[1], gsem[1])
        issue_idx(2, 2)

        def phase(t, b):
            b2 = (b + 2) % 3

            @pl.when(t < ntw)
            def _():
                pltpu.make_async_copy(hp_hbm.at[sidx[b]], rows[b],
                                      gsem[b]).wait()

            @pl.when(t + 2 < ntw)
            def _():
                wait_idx(t + 2, b2)
                pltpu.async_copy(hp_hbm.at[sidx[b2]], rows[b2], gsem[b2])

            @pl.when(t < ntw)
            def _():
                pltpu.sync_copy(rows[b], agg_sh.at[didx[b]], add=True)

            @pl.when(t + 3 < ntw)
            def _():
                issue_idx(t + 3, b)

        @pl.loop(0, (nt + 3) // 3)
        def _(u):
            phase(3 * u, 0)
            phase(3 * u + 1, 1)
            phase(3 * u + 2, 2)

        plsc.subcore_barrier()
        ro = sid * rps
        pltpu.sync_copy(agg_sh.at[pl.ds(ro, rps)],
                        out_hbm.at[cid, pl.ds(ro, rps)])

    return agg(hp, src, dst)


def _deg_col(hist_ref, rb):
    hist = hist_ref[...].astype(jnp.float32)
    ones = jnp.ones((NW, 1), jnp.float32)
    return lax.dot_general(hist, ones, (((0,), (0,)), ((), ()))) + 1.0


def _tc_matmul_scale(x, W, histp):
    n_nodes, d_in = x.shape
    d_out = W.shape[1]
    rb = 1024

    def body(x_ref, w_ref, hist_ref, o_ref):
        deg = _deg_col(hist_ref, rb)
        h = jnp.dot(x_ref[...], w_ref[...], preferred_element_type=jnp.float32)
        o_ref[...] = h * lax.rsqrt(deg)

    return pl.pallas_call(
        body,
        grid=(pl.cdiv(n_nodes, rb),),
        in_specs=[
            pl.BlockSpec((rb, d_in), lambda i: (i, 0)),
            pl.BlockSpec((d_in, d_out), lambda i: (0, 0)),
            pl.BlockSpec((NW, rb), lambda i: (0, i)),
        ],
        out_specs=pl.BlockSpec((rb, d_out), lambda i: (i, 0)),
        out_shape=jax.ShapeDtypeStruct((n_nodes, d_out), jnp.float32),
    )(x, W, histp)


def _tc_combine(aggp, hp, histp, b):
    n_nodes, d_out = hp.shape
    rb = 1024

    def body(p0_ref, p1_ref, hp_ref, hist_ref, b_ref, o_ref):
        deg = _deg_col(hist_ref, rb)
        s = (p0_ref[...] + p1_ref[...] + hp_ref[...]) * lax.rsqrt(deg)
        o_ref[...] = jnp.maximum(s + b_ref[...], 0.0)

    return pl.pallas_call(
        body,
        grid=(pl.cdiv(n_nodes, rb),),
        in_specs=[
            pl.BlockSpec((rb, d_out), lambda i: (i, 0)),
            pl.BlockSpec((rb, d_out), lambda i: (i, 0)),
            pl.BlockSpec((rb, d_out), lambda i: (i, 0)),
            pl.BlockSpec((NW, rb), lambda i: (0, i)),
            pl.BlockSpec((1, d_out), lambda i: (0, 0)),
        ],
        out_specs=pl.BlockSpec((rb, d_out), lambda i: (i, 0)),
        out_shape=jax.ShapeDtypeStruct((n_nodes, d_out), jnp.float32),
    )(aggp[0], aggp[1], hp, histp, b.reshape(1, d_out))


def kernel(x, edge_index, W, b):
    n_nodes = x.shape[0]
    n_pad = _pad_rows(n_nodes)
    src = edge_index[0]
    dst = edge_index[1]
    histp = _sc_hist(dst, n_nodes).reshape(NW, n_pad)
    hp = _tc_matmul_scale(x, W, histp)
    aggp = _sc_agg(hp, src, dst)
    return _tc_combine(aggp, hp, histp, b)


# speedup vs baseline: 40.2757x; 1.0760x over previous
import dataclasses
import functools

import jax
import jax.numpy as jnp
from jax import lax
from jax.experimental import pallas as pl
from jax.experimental.pallas import tpu as pltpu
from jax.experimental.pallas import tpu_sc as plsc

NC = 2
NS = 16
NW = NC * NS
LANES = 16
CHUNK = 128


def _sc_mesh():
    return plsc.VectorSubcoreMesh(core_axis_name="c", subcore_axis_name="s")


def _sc_params():
    cp = pltpu.CompilerParams()
    if "needs_layout_passes" in pltpu.CompilerParams.__dataclass_fields__:
        cp = dataclasses.replace(cp, needs_layout_passes=False)
    return cp


def _pad_rows(n_nodes):
    return -(-n_nodes // (NS * 8)) * (NS * 8)


def _sc_hist(dst, n_nodes):
    n_edges = dst.shape[0]
    nchunks = n_edges // CHUNK
    nt = -(-nchunks // NW)
    n_pad = _pad_rows(n_nodes)
    hrows = n_pad // LANES

    @functools.partial(
        pl.kernel,
        out_type=jax.ShapeDtypeStruct((NW, hrows, LANES), jnp.int32),
        mesh=_sc_mesh(),
        scratch_types=[
            pltpu.VMEM((CHUNK,), jnp.int32),
            pltpu.VMEM((CHUNK,), jnp.int32),
            pltpu.VMEM((CHUNK,), jnp.int32),
            pltpu.VMEM((CHUNK,), jnp.int32),
            pltpu.VMEM((hrows, LANES), jnp.int32),
            pltpu.SemaphoreType.DMA,
            pltpu.SemaphoreType.DMA,
            pltpu.SemaphoreType.DMA,
            pltpu.SemaphoreType.DMA,
        ],
        compiler_params=_sc_params(),
    )
    def hist(dst_hbm, out_hbm, i0, i1, i2, i3, hist_v, s0, s1, s2, s3):
        cid = lax.axis_index("c")
        sid = lax.axis_index("s")
        wid = sid * NC + cid
        ntw = (nchunks - wid + NW - 1) // NW
        zero = jnp.zeros((LANES,), jnp.int32)
        idx = (i0, i1, i2, i3)
        sem = (s0, s1, s2, s3)

        @pl.loop(0, hrows)
        def _(r):
            hist_v[r, :] = zero

        def idx_src(t):
            base = pl.multiple_of((wid + NW * t) * CHUNK, CHUNK)
            return dst_hbm.at[pl.ds(base, CHUNK)]

        for b in range(4):
            pltpu.async_copy(idx_src(b), idx[b], sem[b])

        def phase(t, b):
            @pl.when(t < ntw)
            def _():
                pltpu.make_async_copy(idx_src(t), idx[b], sem[b]).wait()
                for k in range(CHUNK // LANES):
                    v = idx[b][pl.ds(k * LANES, LANES)]
                    cnt, last = plsc.scan_count(v)
                    plsc.addupdate_scatter(
                        hist_v, [v >> 4, v & 15], cnt, mask=last)

            @pl.when(t + 4 < ntw)
            def _():
                pltpu.async_copy(idx_src(t + 4), idx[b], sem[b])

        @pl.loop(0, (nt + 3) // 4)
        def _(u):
            for b in range(4):
                phase(4 * u + b, b)

        pltpu.sync_copy(hist_v, out_hbm.at[wid])

    return hist(dst)


def _sc_agg(hp, src, dst):
    n_nodes, d_out = hp.shape
    n_edges = src.shape[0]
    nchunks = n_edges // CHUNK
    nt = -(-nchunks // NW)
    n_acc = -(-n_nodes // 8) * 8
    main = (n_acc // NS) // 8 * 8
    tail = n_acc - (NS - 1) * main
    zcopies = -(-tail // CHUNK)
    assert (NS - 1) * main + zcopies * CHUNK == n_acc and tail % 8 == 0

    @functools.partial(
        pl.kernel,
        out_type=jax.ShapeDtypeStruct((NC, n_acc, d_out), jnp.float32),
        mesh=_sc_mesh(),
        scratch_types=(
            [pltpu.VMEM((CHUNK,), jnp.int32)] * 12
            + [pltpu.VMEM((CHUNK, d_out), jnp.float32)] * 3
            + [pltpu.VMEM_SHARED((n_acc, d_out), jnp.float32)]
            + [pltpu.SemaphoreType.DMA] * 18
        ),
        compiler_params=_sc_params(),
    )
    def agg(hp_hbm, src_hbm, dst_hbm, out_hbm, *scratch):
        sidx = scratch[0:6]
        didx = scratch[6:12]
        rows = scratch[12:15]
        agg_sh = scratch[15]
        isem_s = scratch[16:22]
        isem_d = scratch[22:28]
        gsem = scratch[28:31]
        ssem = scratch[31:34]
        r0v = rows[0]
        cid = lax.axis_index("c")
        sid = lax.axis_index("s")
        wid = sid * NC + cid
        ntw = (nchunks - wid + NW - 1) // NW
        zero = jnp.zeros((LANES,), jnp.float32)

        def chunk_base(t):
            return pl.multiple_of((wid + NW * t) * CHUNK, CHUNK)

        def issue_idx(t, b):
            base = chunk_base(t)
            pltpu.async_copy(src_hbm.at[pl.ds(base, CHUNK)], sidx[b],
                             isem_s[b])
            pltpu.async_copy(dst_hbm.at[pl.ds(base, CHUNK)], didx[b],
                             isem_d[b])

        def wait_idx(t, b):
            base = chunk_base(t)
            pltpu.make_async_copy(src_hbm.at[pl.ds(base, CHUNK)], sidx[b],
                                  isem_s[b]).wait()
            pltpu.make_async_copy(dst_hbm.at[pl.ds(base, CHUNK)], didx[b],
                                  isem_d[b]).wait()

        @pl.loop(0, CHUNK)
        def _(r):
            @pl.loop(0, d_out // LANES)
            def _(j):
                r0v[r, pl.ds(j * LANES, LANES)] = zero

        for k in range(zcopies):
            pltpu.sync_copy(r0v,
                            agg_sh.at[pl.ds(sid * main + k * CHUNK, CHUNK)])
        plsc.subcore_barrier()

        issue_idx(0, 0)
        issue_idx(1, 1)
        issue_idx(2, 2)

        def phase(t, b):
            b3 = b % 3
            p3 = (b + 2) % 3
            p6 = (b + 5) % 6
            f6 = (b + 3) % 6

            @pl.when(jnp.logical_and(t >= 3, t - 3 < ntw))
            def _():
                pltpu.make_async_copy(rows[b3], agg_sh.at[didx[f6]],
                                      ssem[b3]).wait()

            @pl.when(t < ntw)
            def _():
                wait_idx(t, b)
                pltpu.async_copy(hp_hbm.at[sidx[b]], rows[b3], gsem[b3])

            @pl.when(jnp.logical_and(t >= 1, t - 1 < ntw))
            def _():
                pltpu.make_async_copy(hp_hbm.at[sidx[p6]], rows[p3],
                                      gsem[p3]).wait()
                pltpu.async_copy(rows[p3], agg_sh.at[didx[p6]],
                                 ssem[p3], add=True)

            @pl.when(t + 3 < ntw)
            def _():
                issue_idx(t + 3, f6)

        @pl.loop(0, (nt + 8) // 6)
        def _(u):
            for b in range(6):
                phase(6 * u + b, b)

        plsc.subcore_barrier()
        ro = sid * main

        @pl.when(sid < NS - 1)
        def _():
            pltpu.sync_copy(agg_sh.at[pl.ds(ro, main)],
                            out_hbm.at[cid, pl.ds(ro, main)])

        @pl.when(sid == NS - 1)
        def _():
            pltpu.sync_copy(agg_sh.at[pl.ds((NS - 1) * main, tail)],
                            out_hbm.at[cid, pl.ds((NS - 1) * main, tail)])

    return agg(hp, src, dst)


def _deg_col(hist_ref, rb):
    hist = hist_ref[...].astype(jnp.float32)
    ones = jnp.ones((NW, 1), jnp.float32)
    return lax.dot_general(hist, ones, (((0,), (0,)), ((), ()))) + 1.0


def _tc_matmul_scale(x, W, histp):
    n_nodes, d_in = x.shape
    d_out = W.shape[1]
    rb = 1024

    def body(x_ref, w_ref, hist_ref, o_ref):
        deg = _deg_col(hist_ref, rb)
        h = jnp.dot(x_ref[...], w_ref[...], preferred_element_type=jnp.float32)
        o_ref[...] = h * lax.rsqrt(deg)

    return pl.pallas_call(
        body,
        grid=(pl.cdiv(n_nodes, rb),),
        in_specs=[
            pl.BlockSpec((rb, d_in), lambda i: (i, 0)),
            pl.BlockSpec((d_in, d_out), lambda i: (0, 0)),
            pl.BlockSpec((NW, rb), lambda i: (0, i)),
        ],
        out_specs=pl.BlockSpec((rb, d_out), lambda i: (i, 0)),
        out_shape=jax.ShapeDtypeStruct((n_nodes, d_out), jnp.float32),
    )(x, W, histp)


def _tc_combine(aggp, hp, histp, b):
    n_nodes, d_out = hp.shape
    rb = 1024

    def body(p0_ref, p1_ref, hp_ref, hist_ref, b_ref, o_ref):
        deg = _deg_col(hist_ref, rb)
        s = (p0_ref[...] + p1_ref[...] + hp_ref[...]) * lax.rsqrt(deg)
        o_ref[...] = jnp.maximum(s + b_ref[...], 0.0)

    return pl.pallas_call(
        body,
        grid=(pl.cdiv(n_nodes, rb),),
        in_specs=[
            pl.BlockSpec((rb, d_out), lambda i: (i, 0)),
            pl.BlockSpec((rb, d_out), lambda i: (i, 0)),
            pl.BlockSpec((rb, d_out), lambda i: (i, 0)),
            pl.BlockSpec((NW, rb), lambda i: (0, i)),
            pl.BlockSpec((1, d_out), lambda i: (0, 0)),
        ],
        out_specs=pl.BlockSpec((rb, d_out), lambda i: (i, 0)),
        out_shape=jax.ShapeDtypeStruct((n_nodes, d_out), jnp.float32),
    )(aggp[0], aggp[1], hp, histp, b.reshape(1, d_out))


def kernel(x, edge_index, W, b):
    n_nodes = x.shape[0]
    n_pad = _pad_rows(n_nodes)
    src = edge_index[0]
    dst = edge_index[1]
    histp = _sc_hist(dst, n_nodes).reshape(NW, n_pad)
    hp = _tc_matmul_scale(x, W, histp)
    aggp = _sc_agg(hp, src, dst)
    return _tc_combine(aggp, hp, histp, b)


# speedup vs baseline: 46.9053x; 1.1646x over previous
import dataclasses
import functools

import jax
import jax.numpy as jnp
from jax import lax
from jax.experimental import pallas as pl
from jax.experimental.pallas import tpu as pltpu
from jax.experimental.pallas import tpu_sc as plsc

NC = 2
NS = 16
NW = NC * NS
LANES = 16
CHUNK = 128


def _sc_mesh():
    return plsc.VectorSubcoreMesh(core_axis_name="c", subcore_axis_name="s")


def _sc_params():
    cp = pltpu.CompilerParams()
    if "needs_layout_passes" in pltpu.CompilerParams.__dataclass_fields__:
        cp = dataclasses.replace(cp, needs_layout_passes=False)
    return cp


def _pad_rows(n_nodes):
    return -(-n_nodes // (NS * 8)) * (NS * 8)


def _sc_hist(edge_index, n_nodes):
    n_edges = edge_index.shape[1]
    nchunks = n_edges // CHUNK
    nt = -(-nchunks // NW)
    n_pad = _pad_rows(n_nodes)

    @functools.partial(
        pl.kernel,
        out_type=jax.ShapeDtypeStruct((NW, n_pad), jnp.int32),
        mesh=_sc_mesh(),
        scratch_types=[
            pltpu.VMEM((CHUNK,), jnp.int32),
            pltpu.VMEM((CHUNK,), jnp.int32),
            pltpu.VMEM((CHUNK,), jnp.int32),
            pltpu.VMEM((CHUNK,), jnp.int32),
            pltpu.VMEM((n_pad,), jnp.int32),
            pltpu.SemaphoreType.DMA,
            pltpu.SemaphoreType.DMA,
            pltpu.SemaphoreType.DMA,
            pltpu.SemaphoreType.DMA,
        ],
        compiler_params=_sc_params(),
    )
    def hist(ei_hbm, out_hbm, i0, i1, i2, i3, hist_v, s0, s1, s2, s3):
        cid = lax.axis_index("c")
        sid = lax.axis_index("s")
        wid = sid * NC + cid
        ntw = (nchunks - wid + NW - 1) // NW
        zero = jnp.zeros((LANES,), jnp.int32)
        idx = (i0, i1, i2, i3)
        sem = (s0, s1, s2, s3)

        @pl.loop(0, n_pad // LANES)
        def _(r):
            hist_v[pl.ds(r * LANES, LANES)] = zero

        def idx_src(t):
            base = pl.multiple_of((wid + NW * t) * CHUNK, CHUNK)
            return ei_hbm.at[1, pl.ds(base, CHUNK)]

        for b in range(4):
            pltpu.async_copy(idx_src(b), idx[b], sem[b])

        def phase(t, b):
            @pl.when(t < ntw)
            def _():
                pltpu.make_async_copy(idx_src(t), idx[b], sem[b]).wait()
                for k in range(CHUNK // LANES):
                    v = idx[b][pl.ds(k * LANES, LANES)]
                    cnt, last = plsc.scan_count(v)
                    plsc.addupdate_scatter(hist_v, [v], cnt, mask=last)

            @pl.when(t + 4 < ntw)
            def _():
                pltpu.async_copy(idx_src(t + 4), idx[b], sem[b])

        @pl.loop(0, (nt + 3) // 4)
        def _(u):
            for b in range(4):
                phase(4 * u + b, b)

        pltpu.sync_copy(hist_v, out_hbm.at[wid])

    return hist(edge_index)


def _sc_agg(hp, edge_index):
    n_nodes, d_out = hp.shape
    n_edges = edge_index.shape[1]
    nchunks = n_edges // CHUNK
    nt = -(-nchunks // NW)
    n_acc = -(-n_nodes // 8) * 8
    main = (n_acc // NS) // 8 * 8
    tail = n_acc - (NS - 1) * main
    zcopies = -(-tail // CHUNK)
    assert (NS - 1) * main + zcopies * CHUNK == n_acc and tail % 8 == 0

    @functools.partial(
        pl.kernel,
        out_type=jax.ShapeDtypeStruct((NC, n_acc, d_out), jnp.float32),
        mesh=_sc_mesh(),
        scratch_types=(
            [pltpu.VMEM((CHUNK,), jnp.int32)] * 12
            + [pltpu.VMEM((CHUNK, d_out), jnp.float32)] * 3
            + [pltpu.VMEM_SHARED((n_acc, d_out), jnp.float32)]
            + [pltpu.SemaphoreType.DMA] * 18
        ),
        compiler_params=_sc_params(),
    )
    def agg(hp_hbm, ei_hbm, out_hbm, *scratch):
        sidx = scratch[0:6]
        didx = scratch[6:12]
        rows = scratch[12:15]
        agg_sh = scratch[15]
        isem_s = scratch[16:22]
        isem_d = scratch[22:28]
        gsem = scratch[28:31]
        ssem = scratch[31:34]
        r0v = rows[0]
        cid = lax.axis_index("c")
        sid = lax.axis_index("s")
        wid = sid * NC + cid
        ntw = (nchunks - wid + NW - 1) // NW
        zero = jnp.zeros((LANES,), jnp.float32)

        def chunk_base(t):
            return pl.multiple_of((wid + NW * t) * CHUNK, CHUNK)

        def issue_idx(t, b):
            base = chunk_base(t)
            pltpu.async_copy(ei_hbm.at[0, pl.ds(base, CHUNK)], sidx[b],
                             isem_s[b])
            pltpu.async_copy(ei_hbm.at[1, pl.ds(base, CHUNK)], didx[b],
                             isem_d[b])

        def wait_idx(t, b):
            base = chunk_base(t)
            pltpu.make_async_copy(ei_hbm.at[0, pl.ds(base, CHUNK)], sidx[b],
                                  isem_s[b]).wait()
            pltpu.make_async_copy(ei_hbm.at[1, pl.ds(base, CHUNK)], didx[b],
                                  isem_d[b]).wait()

        @pl.loop(0, CHUNK)
        def _(r):
            @pl.loop(0, d_out // LANES)
            def _(j):
                r0v[r, pl.ds(j * LANES, LANES)] = zero

        for k in range(zcopies):
            pltpu.sync_copy(r0v,
                            agg_sh.at[pl.ds(sid * main + k * CHUNK, CHUNK)])
        plsc.subcore_barrier()

        issue_idx(0, 0)
        issue_idx(1, 1)
        issue_idx(2, 2)

        def phase(t, b):
            b3 = b % 3
            p3 = (b + 2) % 3
            p6 = (b + 5) % 6
            f6 = (b + 3) % 6

            @pl.when(jnp.logical_and(t >= 3, t - 3 < ntw))
            def _():
                pltpu.make_async_copy(rows[b3], agg_sh.at[didx[f6]],
                                      ssem[b3]).wait()

            @pl.when(t < ntw)
            def _():
                wait_idx(t, b)
                pltpu.async_copy(hp_hbm.at[sidx[b]], rows[b3], gsem[b3])

            @pl.when(jnp.logical_and(t >= 1, t - 1 < ntw))
            def _():
                pltpu.make_async_copy(hp_hbm.at[sidx[p6]], rows[p3],
                                      gsem[p3]).wait()
                pltpu.async_copy(rows[p3], agg_sh.at[didx[p6]],
                                 ssem[p3], add=True)

            @pl.when(t + 3 < ntw)
            def _():
                issue_idx(t + 3, f6)

        @pl.loop(0, (nt + 8) // 6)
        def _(u):
            for b in range(6):
                phase(6 * u + b, b)

        plsc.subcore_barrier()
        ro = sid * main

        @pl.when(sid < NS - 1)
        def _():
            pltpu.sync_copy(agg_sh.at[pl.ds(ro, main)],
                            out_hbm.at[cid, pl.ds(ro, main)])

        @pl.when(sid == NS - 1)
        def _():
            pltpu.sync_copy(agg_sh.at[pl.ds((NS - 1) * main, tail)],
                            out_hbm.at[cid, pl.ds((NS - 1) * main, tail)])

    return agg(hp, edge_index)


def _deg_col(hist_ref, rb):
    hist = hist_ref[...].astype(jnp.float32)
    ones = jnp.ones((NW, 1), jnp.float32)
    return lax.dot_general(hist, ones, (((0,), (0,)), ((), ()))) + 1.0


def _tc_matmul_scale(x, W, histp):
    n_nodes, d_in = x.shape
    d_out = W.shape[1]
    rb = 1024

    def body(x_ref, w_ref, hist_ref, o_ref):
        deg = _deg_col(hist_ref, rb)
        h = jnp.dot(x_ref[...], w_ref[...], preferred_element_type=jnp.float32)
        o_ref[...] = h * lax.rsqrt(deg)

    return pl.pallas_call(
        body,
        grid=(pl.cdiv(n_nodes, rb),),
        in_specs=[
            pl.BlockSpec((rb, d_in), lambda i: (i, 0)),
            pl.BlockSpec((d_in, d_out), lambda i: (0, 0)),
            pl.BlockSpec((NW, rb), lambda i: (0, i)),
        ],
        out_specs=pl.BlockSpec((rb, d_out), lambda i: (i, 0)),
        out_shape=jax.ShapeDtypeStruct((n_nodes, d_out), jnp.float32),
    )(x, W, histp)


def _tc_combine(aggp, hp, histp, b):
    n_nodes, d_out = hp.shape
    rb = 1024

    def body(p0_ref, p1_ref, hp_ref, hist_ref, b_ref, o_ref):
        deg = _deg_col(hist_ref, rb)
        s = (p0_ref[...] + p1_ref[...] + hp_ref[...]) * lax.rsqrt(deg)
        o_ref[...] = jnp.maximum(s + b_ref[...], 0.0)

    return pl.pallas_call(
        body,
        grid=(pl.cdiv(n_nodes, rb),),
        in_specs=[
            pl.BlockSpec((rb, d_out), lambda i: (i, 0)),
            pl.BlockSpec((rb, d_out), lambda i: (i, 0)),
            pl.BlockSpec((rb, d_out), lambda i: (i, 0)),
            pl.BlockSpec((NW, rb), lambda i: (0, i)),
            pl.BlockSpec((1, d_out), lambda i: (0, 0)),
        ],
        out_specs=pl.BlockSpec((rb, d_out), lambda i: (i, 0)),
        out_shape=jax.ShapeDtypeStruct((n_nodes, d_out), jnp.float32),
    )(aggp[0], aggp[1], hp, histp, b.reshape(1, d_out))


def kernel(x, edge_index, W, b):
    n_nodes = x.shape[0]
    histp = _sc_hist(edge_index, n_nodes)
    hp = _tc_matmul_scale(x, W, histp)
    aggp = _sc_agg(hp, edge_index)
    return _tc_combine(aggp, hp, histp, b)


# speedup vs baseline: 47.5157x; 1.0130x over previous
import dataclasses
import functools

import jax
import jax.numpy as jnp
from jax import lax
from jax.experimental import pallas as pl
from jax.experimental.pallas import tpu as pltpu
from jax.experimental.pallas import tpu_sc as plsc

NC = 2
NS = 16
NW = NC * NS
LANES = 16
CHUNK = 128


def _sc_mesh():
    return plsc.VectorSubcoreMesh(core_axis_name="c", subcore_axis_name="s")


def _sc_params():
    cp = pltpu.CompilerParams()
    if "needs_layout_passes" in pltpu.CompilerParams.__dataclass_fields__:
        cp = dataclasses.replace(cp, needs_layout_passes=False)
    return cp


def _pad_rows(n_nodes):
    return -(-n_nodes // (NS * 8)) * (NS * 8)


def _sc_hist(edge_index, n_nodes):
    n_edges = edge_index.shape[1]
    nchunks = n_edges // CHUNK
    n_pad = _pad_rows(n_nodes)
    mainc = nchunks // NW
    nextra = nchunks - NW * mainc
    sbc = 1
    for cand in range(8, 1, -1):
        if mainc % cand == 0:
            sbc = cand
            break
    nsb = mainc // sbc

    @functools.partial(
        pl.kernel,
        out_type=jax.ShapeDtypeStruct((NW, n_pad), jnp.int32),
        mesh=_sc_mesh(),
        scratch_types=[
            pltpu.VMEM((sbc * CHUNK,), jnp.int32),
            pltpu.VMEM((sbc * CHUNK,), jnp.int32),
            pltpu.VMEM((n_pad,), jnp.int32),
            pltpu.SemaphoreType.DMA,
            pltpu.SemaphoreType.DMA,
        ],
        compiler_params=_sc_params(),
    )
    def hist(ei_hbm, out_hbm, i0, i1, hist_v, s0, s1):
        cid = lax.axis_index("c")
        sid = lax.axis_index("s")
        wid = sid * NC + cid
        zero = jnp.zeros((LANES,), jnp.int32)
        idx = (i0, i1)
        sem = (s0, s1)

        @pl.loop(0, n_pad // LANES)
        def _(r):
            hist_v[pl.ds(r * LANES, LANES)] = zero

        def sb_src(g):
            base = pl.multiple_of((wid * mainc + g * sbc) * CHUNK,
                                  sbc * CHUNK)
            return ei_hbm.at[1, pl.ds(base, sbc * CHUNK)]

        def count16(v):
            cnt, last = plsc.scan_count(v)
            plsc.addupdate_scatter(hist_v, [v], cnt, mask=last)

        pltpu.async_copy(sb_src(0), idx[0], sem[0])

        def phase(g, b):
            @pl.when(g < nsb)
            def _():
                pltpu.make_async_copy(sb_src(g), idx[b], sem[b]).wait()

                @pl.when(g + 1 < nsb)
                def _():
                    pltpu.async_copy(sb_src(g + 1), idx[1 - b], sem[1 - b])

                for k in range(sbc * CHUNK // LANES):
                    count16(idx[b][pl.ds(k * LANES, LANES)])

        @pl.loop(0, (nsb + 1) // 2)
        def _(u):
            phase(2 * u, 0)
            phase(2 * u + 1, 1)

        if nextra:
            @pl.when(wid < nextra)
            def _():
                base = pl.multiple_of((NW * mainc + wid) * CHUNK, CHUNK)
                pltpu.sync_copy(ei_hbm.at[1, pl.ds(base, CHUNK)],
                                i0.at[pl.ds(0, CHUNK)])
                for k in range(CHUNK // LANES):
                    count16(i0[pl.ds(k * LANES, LANES)])

        pltpu.sync_copy(hist_v, out_hbm.at[wid])

    return hist(edge_index)


def _sc_agg(hp, edge_index):
    n_nodes, d_out = hp.shape
    n_edges = edge_index.shape[1]
    nchunks = n_edges // CHUNK
    nt = -(-nchunks // NW)
    n_acc = -(-n_nodes // 8) * 8
    main = (n_acc // NS) // 8 * 8
    tail = n_acc - (NS - 1) * main
    zcopies = -(-tail // CHUNK)
    assert (NS - 1) * main + zcopies * CHUNK == n_acc and tail % 8 == 0

    @functools.partial(
        pl.kernel,
        out_type=jax.ShapeDtypeStruct((NC, n_acc, d_out), jnp.float32),
        mesh=_sc_mesh(),
        scratch_types=(
            [pltpu.VMEM((CHUNK,), jnp.int32)] * 12
            + [pltpu.VMEM((CHUNK, d_out), jnp.float32)] * 3
            + [pltpu.VMEM_SHARED((n_acc, d_out), jnp.float32)]
            + [pltpu.SemaphoreType.DMA] * 18
        ),
        compiler_params=_sc_params(),
    )
    def agg(hp_hbm, ei_hbm, out_hbm, *scratch):
        sidx = scratch[0:6]
        didx = scratch[6:12]
        rows = scratch[12:15]
        agg_sh = scratch[15]
        isem_s = scratch[16:22]
        isem_d = scratch[22:28]
        gsem = scratch[28:31]
        ssem = scratch[31:34]
        r0v = rows[0]
        cid = lax.axis_index("c")
        sid = lax.axis_index("s")
        wid = sid * NC + cid
        ntw = (nchunks - wid + NW - 1) // NW
        zero = jnp.zeros((LANES,), jnp.float32)

        def chunk_base(t):
            return pl.multiple_of((wid + NW * t) * CHUNK, CHUNK)

        def issue_idx(t, b):
            base = chunk_base(t)
            pltpu.async_copy(ei_hbm.at[0, pl.ds(base, CHUNK)], sidx[b],
                             isem_s[b])
            pltpu.async_copy(ei_hbm.at[1, pl.ds(base, CHUNK)], didx[b],
                             isem_d[b])

        def wait_idx(t, b):
            base = chunk_base(t)
            pltpu.make_async_copy(ei_hbm.at[0, pl.ds(base, CHUNK)], sidx[b],
                                  isem_s[b]).wait()
            pltpu.make_async_copy(ei_hbm.at[1, pl.ds(base, CHUNK)], didx[b],
                                  isem_d[b]).wait()

        @pl.loop(0, CHUNK)
        def _(r):
            @pl.loop(0, d_out // LANES)
            def _(j):
                r0v[r, pl.ds(j * LANES, LANES)] = zero

        for k in range(zcopies):
            pltpu.sync_copy(r0v,
                            agg_sh.at[pl.ds(sid * main + k * CHUNK, CHUNK)])
        plsc.subcore_barrier()

        issue_idx(0, 0)
        issue_idx(1, 1)
        issue_idx(2, 2)

        def phase(t, b):
            b3 = b % 3
            p3 = (b + 2) % 3
            p6 = (b + 5) % 6
            f6 = (b + 3) % 6

            @pl.when(jnp.logical_and(t >= 3, t - 3 < ntw))
            def _():
                pltpu.make_async_copy(rows[b3], agg_sh.at[didx[f6]],
                                      ssem[b3]).wait()

            @pl.when(t < ntw)
            def _():
                wait_idx(t, b)
                pltpu.async_copy(hp_hbm.at[sidx[b]], rows[b3], gsem[b3])

            @pl.when(jnp.logical_and(t >= 1, t - 1 < ntw))
            def _():
                pltpu.make_async_copy(hp_hbm.at[sidx[p6]], rows[p3],
                                      gsem[p3]).wait()
                pltpu.async_copy(rows[p3], agg_sh.at[didx[p6]],
                                 ssem[p3], add=True)

            @pl.when(t + 3 < ntw)
            def _():
                issue_idx(t + 3, f6)

        @pl.loop(0, (nt + 8) // 6)
        def _(u):
            for b in range(6):
                phase(6 * u + b, b)

        plsc.subcore_barrier()
        ro = sid * main

        @pl.when(sid < NS - 1)
        def _():
            pltpu.sync_copy(agg_sh.at[pl.ds(ro, main)],
                            out_hbm.at[cid, pl.ds(ro, main)])

        @pl.when(sid == NS - 1)
        def _():
            pltpu.sync_copy(agg_sh.at[pl.ds((NS - 1) * main, tail)],
                            out_hbm.at[cid, pl.ds((NS - 1) * main, tail)])

    return agg(hp, edge_index)


def _deg_col(hist_ref, rb):
    hist = hist_ref[...].astype(jnp.float32)
    ones = jnp.ones((NW, 1), jnp.float32)
    return lax.dot_general(hist, ones, (((0,), (0,)), ((), ()))) + 1.0


def _tc_matmul_scale(x, W, histp):
    n_nodes, d_in = x.shape
    d_out = W.shape[1]
    rb = 1024

    def body(x_ref, w_ref, hist_ref, o_ref):
        deg = _deg_col(hist_ref, rb)
        h = jnp.dot(x_ref[...], w_ref[...], preferred_element_type=jnp.float32)
        o_ref[...] = h * lax.rsqrt(deg)

    return pl.pallas_call(
        body,
        grid=(pl.cdiv(n_nodes, rb),),
        in_specs=[
            pl.BlockSpec((rb, d_in), lambda i: (i, 0)),
            pl.BlockSpec((d_in, d_out), lambda i: (0, 0)),
            pl.BlockSpec((NW, rb), lambda i: (0, i)),
        ],
        out_specs=pl.BlockSpec((rb, d_out), lambda i: (i, 0)),
        out_shape=jax.ShapeDtypeStruct((n_nodes, d_out), jnp.float32),
    )(x, W, histp)


def _tc_combine(aggp, hp, histp, b):
    n_nodes, d_out = hp.shape
    rb = 1024

    def body(p0_ref, p1_ref, hp_ref, hist_ref, b_ref, o_ref):
        deg = _deg_col(hist_ref, rb)
        s = (p0_ref[...] + p1_ref[...] + hp_ref[...]) * lax.rsqrt(deg)
        o_ref[...] = jnp.maximum(s + b_ref[...], 0.0)

    return pl.pallas_call(
        body,
        grid=(pl.cdiv(n_nodes, rb),),
        in_specs=[
            pl.BlockSpec((rb, d_out), lambda i: (i, 0)),
            pl.BlockSpec((rb, d_out), lambda i: (i, 0)),
            pl.BlockSpec((rb, d_out), lambda i: (i, 0)),
            pl.BlockSpec((NW, rb), lambda i: (0, i)),
            pl.BlockSpec((1, d_out), lambda i: (0, 0)),
        ],
        out_specs=pl.BlockSpec((rb, d_out), lambda i: (i, 0)),
        out_shape=jax.ShapeDtypeStruct((n_nodes, d_out), jnp.float32),
    )(aggp[0], aggp[1], hp, histp, b.reshape(1, d_out))


def kernel(x, edge_index, W, b):
    n_nodes = x.shape[0]
    histp = _sc_hist(edge_index, n_nodes)
    hp = _tc_matmul_scale(x, W, histp)
    aggp = _sc_agg(hp, edge_index)
    return _tc_combine(aggp, hp, histp, b)
